# Initial kernel scaffold; baseline (speedup 1.0000x reference)
import jax
import jax.numpy as jnp
from jax.experimental import pallas as pl


def kernel(x, table):
    raise NotImplementedError("write your pallas kernel here")


# speedup vs baseline: 1.0603x; 1.0603x over previous
import functools

import jax
import jax.numpy as jnp
import numpy as np
from jax import lax
from jax.experimental import pallas as pl
from jax.experimental.pallas import tpu as pltpu
from jax.experimental.pallas import tpu_sc as plsc

VOCAB = 1000000
D_MODEL = 128
B = 4
L = 2048
PE_LEN = 2048
SCALE = float(np.sqrt(np.float64(D_MODEL)))

NUM_WORKERS = 32
ROWS_PER_W = (B * L) // NUM_WORKERS
GATHER_CHUNK = 128
N_CHUNKS = ROWS_PER_W // GATHER_CHUNK


def _pe_table() -> np.ndarray:
    depth = D_MODEL / 2
    positions = np.arange(PE_LEN)[:, np.newaxis]
    depths = np.arange(depth)[np.newaxis, :] / depth
    angle_rads = positions * (1 / 10000**depths)
    return np.concatenate(
        [np.sin(angle_rads), np.cos(angle_rads)], axis=-1
    ).astype(np.float32)


_PE = _pe_table()


def _sc_body(x_hbm, pe_hbm, table_hbm, out_hbm, idx_v, rows_v, pe_v, sem, sem_pe):
    wid = lax.axis_index("s") * 2 + lax.axis_index("c")
    base = wid * ROWS_PER_W
    pos_base = lax.rem(base, L)

    pltpu.sync_copy(x_hbm.at[wid], idx_v)

    h_pe = pltpu.async_copy(pe_hbm.at[pl.ds(pos_base, ROWS_PER_W)], pe_v, sem_pe)
    handles = []
    for j in range(N_CHUNKS):
        handles.append(
            pltpu.async_copy(
                table_hbm.at[idx_v.at[j]],
                rows_v.at[pl.ds(j * GATHER_CHUNK, GATHER_CHUNK)],
                sem,
            )
        )
    h_pe.wait()
    for h in handles:
        h.wait()

    def fma_row(i, carry):
        for c in range(D_MODEL // 16):
            sl = pl.ds(c * 16, 16)
            rows_v[i, sl] = rows_v[i, sl] * SCALE + pe_v[i, sl]
        return carry

    lax.fori_loop(0, ROWS_PER_W, fma_row, 0)

    pltpu.sync_copy(rows_v, out_hbm.at[pl.ds(base, ROWS_PER_W)])


def kernel(x, table):
    x_grp = x.reshape(NUM_WORKERS, N_CHUNKS, GATHER_CHUNK)
    pe = jnp.asarray(_PE)

    sc_call = functools.partial(
        pl.kernel,
        out_type=jax.ShapeDtypeStruct((B * L, D_MODEL), jnp.float32),
        mesh=plsc.VectorSubcoreMesh(core_axis_name="c", subcore_axis_name="s"),
        scratch_types=[
            pltpu.VMEM((N_CHUNKS, GATHER_CHUNK), jnp.int32),
            pltpu.VMEM((ROWS_PER_W, D_MODEL), jnp.float32),
            pltpu.VMEM((ROWS_PER_W, D_MODEL), jnp.float32),
            pltpu.SemaphoreType.DMA,
            pltpu.SemaphoreType.DMA,
        ],
    )(_sc_body)

    out = sc_call(x_grp, pe, table)
    return out.reshape(B, L, D_MODEL)


# speedup vs baseline: 1.1205x; 1.0567x over previous
import functools

import jax
import jax.numpy as jnp
import numpy as np
from jax import lax
from jax.experimental import pallas as pl
from jax.experimental.pallas import tpu as pltpu
from jax.experimental.pallas import tpu_sc as plsc

VOCAB = 1000000
D_MODEL = 128
B = 4
L = 2048
PE_LEN = 2048
SCALE = float(np.sqrt(np.float64(D_MODEL)))

NUM_WORKERS = 32
ROWS_PER_W = (B * L) // NUM_WORKERS
CHUNK = 128
N_CHUNKS = ROWS_PER_W // CHUNK
LANES = 16


def _pe_table() -> np.ndarray:
    depth = D_MODEL / 2
    positions = np.arange(PE_LEN)[:, np.newaxis]
    depths = np.arange(depth)[np.newaxis, :] / depth
    angle_rads = positions * (1 / 10000**depths)
    return np.concatenate(
        [np.sin(angle_rads), np.cos(angle_rads)], axis=-1
    ).astype(np.float32)


_PE = _pe_table()


def _sc_body(x_hbm, pe_hbm, table_hbm, out_hbm,
             idx_v, acc_v, g_v, sem_g0, sem_g1, sem_pe, sem_st):
    wid = lax.axis_index("s") * 2 + lax.axis_index("c")
    b = wid // (L // ROWS_PER_W)
    l0 = lax.rem(wid, L // ROWS_PER_W) * ROWS_PER_W

    pltpu.sync_copy(x_hbm.at[b, pl.ds(l0, ROWS_PER_W)], idx_v)

    h_pe = pltpu.async_copy(pe_hbm.at[pl.ds(l0, ROWS_PER_W)], acc_v, sem_pe)
    h_g0 = pltpu.async_copy(
        table_hbm.at[idx_v.at[pl.ds(0, CHUNK)]],
        g_v.at[pl.ds(0, CHUNK)], sem_g0)
    h_g1 = pltpu.async_copy(
        table_hbm.at[idx_v.at[pl.ds(CHUNK, CHUNK)]],
        g_v.at[pl.ds(CHUNK, CHUNK)], sem_g1)

    def fma_row(i, carry):
        for c in range(D_MODEL // LANES):
            sl = pl.ds(c * LANES, LANES)
            plsc.addupdate(acc_v.at[i, sl], g_v[i, sl] * SCALE)
        return carry

    h_pe.wait()
    h_g0.wait()
    lax.fori_loop(0, CHUNK, fma_row, 0)
    h_st0 = pltpu.async_copy(
        acc_v.at[pl.ds(0, CHUNK)], out_hbm.at[b, pl.ds(l0, CHUNK)], sem_st)
    h_g1.wait()
    lax.fori_loop(CHUNK, ROWS_PER_W, fma_row, 0)
    h_st0.wait()
    pltpu.sync_copy(
        acc_v.at[pl.ds(CHUNK, CHUNK)], out_hbm.at[b, pl.ds(l0 + CHUNK, CHUNK)])


def kernel(x, table):
    pe = jnp.asarray(_PE)

    sc_call = functools.partial(
        pl.kernel,
        out_type=jax.ShapeDtypeStruct((B, L, D_MODEL), jnp.float32),
        mesh=plsc.VectorSubcoreMesh(core_axis_name="c", subcore_axis_name="s"),
        scratch_types=[
            pltpu.VMEM((ROWS_PER_W,), jnp.int32),
            pltpu.VMEM((ROWS_PER_W, D_MODEL), jnp.float32),
            pltpu.VMEM((ROWS_PER_W, D_MODEL), jnp.float32),
            pltpu.SemaphoreType.DMA,
            pltpu.SemaphoreType.DMA,
            pltpu.SemaphoreType.DMA,
            pltpu.SemaphoreType.DMA,
        ],
    )(_sc_body)

    return sc_call(x, pe, table)
